# baseline (speedup 1.0000x reference)
import jax
import jax.numpy as jnp
from jax import lax
from jax.experimental import pallas as pl
from jax.experimental.pallas import tpu as pltpu
from jax.experimental.pallas import tpu_sc as plsc

_B, _T, _NEOF, _EMB = 4, 2048, 128, 32
_NW = 32
_ROWS = (_B * _T) // _NW
_L = 16
_NJ = _NEOF // _L
_NC = 4
_CROWS = _ROWS // _NC
_WOFF = 16


def _sc_body(x_hbm, tt_hbm, ft_hbm, w_hbm, b_hbm, out_hbm,
             x_v, tt_v, ft_v, w_v, fb_v, tp_v, in_sems, out_sems):
    wid = lax.axis_index("s") * 2 + lax.axis_index("c")
    base = wid * _ROWS
    t0 = base % _T

    pltpu.sync_copy(tt_hbm.at[pl.ds(t0, _ROWS)], tt_v)
    pltpu.sync_copy(ft_hbm, ft_v)
    pltpu.sync_copy(w_hbm, w_v.at[pl.ds(_WOFF, _EMB)])
    pltpu.sync_copy(b_hbm, w_v.at[pl.ds(8, 1)])
    in_handles = [
        pltpu.async_copy(x_hbm.at[pl.ds(base + c * _CROWS, _CROWS)],
                         x_v.at[pl.ds(c * _CROWS, _CROWS)], in_sems[c])
        for c in range(_NC)
    ]

    iota = lax.iota(jnp.int32, _L)
    lane0 = iota == 0
    w0 = w_v[pl.ds(_WOFF, _L)]
    w1 = w_v[pl.ds(_WOFF + _L, _L)]
    b_s = jnp.sum(jnp.where(lane0, w_v[pl.ds(8, _L)], 0.0))

    def fdot(i, carry):
        s = (jnp.sum(ft_v[i, pl.ds(0, _L)] * w0) +
             jnp.sum(ft_v[i, pl.ds(_L, _L)] * w1) + b_s)
        plsc.store_scatter(fb_v, [jnp.full((_L,), i, jnp.int32)],
                           jnp.zeros((_L,), jnp.float32) + s, mask=lane0)
        return carry

    lax.fori_loop(0, _NEOF, fdot, 0)

    def tdot(i, carry):
        s = (jnp.sum(tt_v[i, pl.ds(0, _L)] * w0) +
             jnp.sum(tt_v[i, pl.ds(_L, _L)] * w1))
        plsc.store_scatter(tp_v, [jnp.full((_L,), i, jnp.int32)],
                           jnp.zeros((_L,), jnp.float32) + s, mask=lane0)
        return carry

    lax.fori_loop(0, _ROWS, tdot, 0)
    facc = [fb_v[pl.ds(j * _L, _L)] for j in range(_NJ)]

    out_handles = []
    for c in range(_NC):
        in_handles[c].wait()
        r0 = c * _CROWS

        def row_step(i, carry, r0=r0):
            r = r0 + i
            tpb = plsc.load_gather(tp_v, [jnp.full((_L,), r, jnp.int32)])
            for j in range(_NJ):
                sl = pl.ds(j * _L, _L)
                x_v[r, sl] = x_v[r, sl] + (facc[j] + tpb)
            return carry

        lax.fori_loop(0, _CROWS, row_step, 0)
        out_handles.append(
            pltpu.async_copy(x_v.at[pl.ds(r0, _CROWS)],
                             out_hbm.at[pl.ds(base + r0, _CROWS)],
                             out_sems[c]))
    for h in out_handles:
        h.wait()


def kernel(X, time_table, feature_table, W, b):
    Xf = X.reshape(_B * _T, _NEOF)

    mesh = plsc.VectorSubcoreMesh(core_axis_name="c", subcore_axis_name="s")
    run = pl.kernel(
        _sc_body,
        mesh=mesh,
        out_type=jax.ShapeDtypeStruct((_B * _T, _NEOF), jnp.float32),
        scratch_types=[
            pltpu.VMEM((_ROWS, _NEOF), jnp.float32),
            pltpu.VMEM((_ROWS, _EMB), jnp.float32),
            pltpu.VMEM((_NEOF, _EMB), jnp.float32),
            pltpu.VMEM((3 * _L,), jnp.float32),
            pltpu.VMEM((_NEOF,), jnp.float32),
            pltpu.VMEM((_ROWS,), jnp.float32),
            [pltpu.SemaphoreType.DMA] * _NC,
            [pltpu.SemaphoreType.DMA] * _NC,
        ],
        compiler_params=pltpu.CompilerParams(
            needs_layout_passes=False, use_tc_tiling_on_sc=True),
    )
    out = run(Xf, time_table, feature_table, W.reshape(_EMB), b)
    return out.reshape(_B, _T, _NEOF)
